# baseline (speedup 1.0000x reference)
import jax
import jax.numpy as jnp
from jax import lax
from jax.experimental import pallas as pl
from jax.experimental.pallas import tpu as pltpu
from jax.experimental.pallas import tpu_sc as plsc

_MAX_LEN = 200
_DIM = 64
_BATCH = 4096

_NC, _NS, _L = 2, 16, 16
_NW = _NC * _NS
_ROWS = _BATCH * _MAX_LEN
_RPW = _ROWS // _NW
_CHUNK = 128
_NCHUNK = _RPW // _CHUNK


def _pe_doubled():
    pos = jnp.arange(_MAX_LEN, dtype=jnp.float32).reshape(-1, 1)
    dim = jnp.arange(_DIM, dtype=jnp.float32).reshape(1, -1)
    phase = pos / (10000.0 ** (dim / _DIM))
    pe = jnp.where((jnp.arange(_DIM) % 2) == 0, jnp.sin(phase), jnp.cos(phase))
    pe = jnp.concatenate([pe, pe], axis=0)
    return jnp.pad(pe, ((0, 0), (0, _DIM)))


_NBUF = 4


def _body(tok_hbm, pe_hbm, table_hbm, out_hbm, idx_v, pe_sh, *rest):
    sid = lax.axis_index("s")
    wid = sid * _NC + lax.axis_index("c")
    chunk0 = wid * _NCHUNK
    row0 = wid * _RPW

    pltpu.sync_copy(tok_hbm.at[pl.ds(chunk0, _NCHUNK)], idx_v)

    @pl.when(sid == 0)
    def _():
        pltpu.sync_copy(pe_hbm, pe_sh)

    plsc.subcore_barrier()

    bufs = rest[:_NBUF]
    obufs = rest[_NBUF:_NBUF + 2]
    gsems = rest[_NBUF + 2:2 * _NBUF + 2]
    ssems = rest[2 * _NBUF + 2:2 * _NBUF + 4]

    def _pe_fill(c, b):
        pos0 = lax.rem(c * _CHUNK, _MAX_LEN)
        pltpu.sync_copy(pe_sh.at[pl.ds(pos0, _CHUNK)], bufs[b])

    def _gather_start(c, b):
        pltpu.async_copy(table_hbm.at[idx_v.at[c]], bufs[b], gsems[b], add=True)

    def _gather_wait(c, b):
        pltpu.make_async_copy(table_hbm.at[idx_v.at[c]], bufs[b], gsems[b]).wait()

    def _scatter_start(c, o):
        pltpu.async_copy(
            obufs[o], out_hbm.at[pl.ds(row0 + c * _CHUNK, _CHUNK)], ssems[o]
        )

    def _scatter_wait(c, o):
        pltpu.make_async_copy(
            obufs[o], out_hbm.at[pl.ds(row0 + c * _CHUNK, _CHUNK)], ssems[o]
        ).wait()

    def _copy_out(b, o):
        buf, ob = bufs[b], obufs[o]

        @plsc.parallel_loop(0, _CHUNK, unroll=8)
        def _(r):
            for j in range(_DIM // _L):
                sl = pl.ds(j * _L, _L)
                ob[r, sl] = buf[r, sl]

    for c in range(_NBUF - 1):
        _pe_fill(c, c)
        _gather_start(c, c)

    def outer(g, _):
        for b in range(_NBUF):
            c = g * _NBUF + b
            o = b % 2
            _gather_wait(c, b)

            @pl.when(c >= 2)
            def _():
                _scatter_wait(c - 2, o)

            _copy_out(b, o)
            _scatter_start(c, o)

            cn = c + (_NBUF - 1)
            bn = (b + _NBUF - 1) % _NBUF

            @pl.when(cn < _NCHUNK)
            def _():
                _pe_fill(cn, bn)
                _gather_start(cn, bn)

        return 0

    lax.fori_loop(0, _NCHUNK // _NBUF, outer, 0)

    _scatter_wait(_NCHUNK - 2, 0)
    _scatter_wait(_NCHUNK - 1, 1)


_mesh = plsc.VectorSubcoreMesh(core_axis_name="c", subcore_axis_name="s")

_sc_call = pl.kernel(
    _body,
    out_type=jax.ShapeDtypeStruct((_ROWS, _DIM), jnp.float32),
    mesh=_mesh,
    scratch_types=[
        pltpu.VMEM((_NCHUNK, _CHUNK), jnp.int32),
        pltpu.VMEM_SHARED((2 * _MAX_LEN, 2 * _DIM), jnp.float32),
        *([pltpu.VMEM((_CHUNK, 2 * _DIM), jnp.float32)] * _NBUF),
        *([pltpu.VMEM((_CHUNK, _DIM), jnp.float32)] * 2),
        *([pltpu.SemaphoreType.DMA] * _NBUF),
        *([pltpu.SemaphoreType.DMA] * 2),
    ],
    compiler_params=pltpu.CompilerParams(use_tc_tiling_on_sc=True),
)


def kernel(tokens, table):
    tok = tokens.astype(jnp.int32).reshape(_ROWS // _CHUNK, _CHUNK)
    pe = _pe_doubled()
    table_p = jnp.pad(table, ((0, 0), (0, _DIM)))
    out = _sc_call(tok, pe, table_p)
    return out.reshape(_BATCH, _MAX_LEN, _DIM)
